# Initial kernel scaffold; baseline (speedup 1.0000x reference)
import jax
import jax.numpy as jnp
from jax.experimental import pallas as pl


def kernel(means3D, primitive_confidences, feature_table, linear_weights, linear_bias, viewmatrix, projmatrix, cam_pos, bg_color):
    raise NotImplementedError("write your pallas kernel here")


# speedup vs baseline: 6.0979x; 6.0979x over previous
import numpy as np
import jax
import jax.numpy as jnp
from jax import lax
from jax.experimental import pallas as pl
from jax.experimental.pallas import tpu as pltpu
from jax.experimental.pallas import tpu_sc as plsc

H = 512
W = 512
L = 16
F = 2
T_PER = 16384
NEAR = 0.1
FAR = 100.0
OCC_TH = 0.5
PRIME1 = int(np.array(2654435761, dtype=np.uint32).view(np.int32))
PRIME2 = 805459861
RES_F = [float(int(np.floor(16.0 * (1.3819 ** l)))) for l in range(L)]

NC = 2
NS = 16
NPIX = H * W
HALF = NPIX // 2
QUART = NPIX // 4
ZONE = NPIX // 8
NZONE = 4
PAD = 8

P_TOTAL = 65536
CH_A = 4096
CH_P = 1024
CH_C = P_TOTAL // NS
CH_D = 512
ZQ = ZONE // NS
SUBPX = 1024

_MESH = plsc.VectorSubcoreMesh(
    core_axis_name="c", subcore_axis_name="s", num_cores=NC, num_subcores=NS)
_SC_PARAMS = pltpu.CompilerParams(needs_layout_passes=False)


def _sigmoid(x):
    return 1.0 / (1.0 + jnp.exp(-x))


def _bf16_round(v):
    b = plsc.bitcast(v, jnp.int32)
    b = b + 32767 + ((b >> 16) & 1)
    return plsc.bitcast(b & jnp.int32(-65536), jnp.float32)


def _proj_body(meansT, confT, mrep_hbm, pix_hbm, wbase_hbm,
               mb, cb, mrep_v, pixb, wbb):
    c = lax.axis_index("c")
    s = lax.axis_index("s")
    wid = c * NS + s
    pltpu.sync_copy(mrep_hbm, mrep_v)

    def msl(k):
        return mrep_v[pl.ds(k * 16, 16)]

    def chunk(g, carry):
        base = wid * (P_TOTAL // 32) + g * CH_P
        pltpu.sync_copy(meansT.at[:, pl.ds(base, CH_P)], mb)
        pltpu.sync_copy(confT.at[:, pl.ds(base, CH_P)], cb)

        def vec(i, carry2):
            sl = pl.ds(i * 16, 16)
            x = _bf16_round(mb[0, sl])
            y = _bf16_round(mb[1, sl])
            z = _bf16_round(mb[2, sl])
            depth = msl(0) * x + msl(1) * y + msl(2) * z + msl(3)
            ph0 = msl(4) * x + msl(5) * y + msl(6) * z + msl(7)
            ph1 = msl(8) * x + msl(9) * y + msl(10) * z + msl(11)
            ph3 = msl(12) * x + msl(13) * y + msl(14) * z + msl(15)
            wclip = jnp.maximum(ph3, 1e-6)
            px = ((ph0 / wclip + 1.0) * W - 1.0) * 0.5
            py = ((ph1 / wclip + 1.0) * H - 1.0) * 0.5
            valid = ((depth > NEAR) & (depth < FAR)
                     & (px >= 0) & (px < W) & (py >= 0) & (py < H))
            occ = cb[0, sl]
            for r in range(1, 27):
                occ = occ + cb[r, sl]
            occ = occ / 27.0
            gate = _sigmoid((occ - OCC_TH) * 10.0)
            trans = jnp.exp(-depth / FAR)
            validf = valid.astype(jnp.float32)
            wbase = occ * gate * trans * validf
            pxi = jnp.clip(px.astype(jnp.int32), 0, W - 1)
            pyi = jnp.clip(py.astype(jnp.int32), 0, H - 1)
            pix = jnp.where(valid, pyi * W + pxi, 0)
            pixb[sl] = pix
            wbb[sl] = wbase
            return carry2

        lax.fori_loop(0, CH_P // 16, vec, 0)
        pltpu.sync_copy(pixb, pix_hbm.at[pl.ds(base, CH_P)])
        pltpu.sync_copy(wbb, wbase_hbm.at[pl.ds(base, CH_P)])
        return carry

    lax.fori_loop(0, (P_TOTAL // 32) // CH_P, chunk, 0)


_proj_call = pl.kernel(
    _proj_body,
    out_type=[
        jax.ShapeDtypeStruct((P_TOTAL,), jnp.int32),
        jax.ShapeDtypeStruct((P_TOTAL,), jnp.float32),
    ],
    mesh=_MESH,
    compiler_params=_SC_PARAMS,
    scratch_types=[
        pltpu.VMEM((3, CH_P), jnp.float32),
        pltpu.VMEM((27, CH_P), jnp.float32),
        pltpu.VMEM((256,), jnp.float32),
        pltpu.VMEM((CH_P,), jnp.int32),
        pltpu.VMEM((CH_P,), jnp.float32),
    ],
)


def _gather_body(meansT, table, featsT, sub_v, xs_v, ys_v, zs_v, f0_v, f1_v):
    c = lax.axis_index("c")
    s = lax.axis_index("s")
    level = s
    halfbase = c * (P_TOTAL // 2)

    lvl = jnp.full((16,), level, jnp.int32)
    res = jnp.zeros((16,), jnp.float32)
    for l in range(L):
        res = jnp.where(lvl == l, jnp.float32(RES_F[l]), res)

    pltpu.sync_copy(table.at[pl.ds(level * (T_PER * F), T_PER * F)], sub_v)

    def chunk_body(g, carry):
        base = halfbase + g * CH_A
        pltpu.sync_copy(meansT.at[pl.ds(0, 1), pl.ds(base, CH_A)], xs_v)
        pltpu.sync_copy(meansT.at[pl.ds(1, 1), pl.ds(base, CH_A)], ys_v)
        pltpu.sync_copy(meansT.at[pl.ds(2, 1), pl.ds(base, CH_A)], zs_v)

        def vec_body(i, carry2):
            sl = pl.ds(i * 16, 16)
            x01 = jnp.clip((xs_v[0, sl] - (-1.5)) / 3.0, 0.0, 1.0 - 1e-6)
            y01 = jnp.clip((ys_v[0, sl] - (-1.5)) / 3.0, 0.0, 1.0 - 1e-6)
            z01 = jnp.clip((zs_v[0, sl] - 0.0) / 8.0, 0.0, 1.0 - 1e-6)
            x = x01 * res
            y = y01 * res
            z = z01 * res
            xi = x.astype(jnp.int32)
            yi = y.astype(jnp.int32)
            zi = z.astype(jnp.int32)
            wx = x - xi.astype(jnp.float32)
            wy = y - yi.astype(jnp.float32)
            wz = z - zi.astype(jnp.float32)
            hy = (yi * PRIME1, yi * PRIME1 + PRIME1)
            hz = (zi * PRIME2, zi * PRIME2 + PRIME2)
            hx = (xi, xi + 1)
            wxs = (1.0 - wx, wx)
            wys = (1.0 - wy, wy)
            wzs = (1.0 - wz, wz)
            acc0 = jnp.zeros((16,), jnp.float32)
            acc1 = jnp.zeros((16,), jnp.float32)
            for dx in (0, 1):
                for dy in (0, 1):
                    hxy = hx[dx] ^ hy[dy]
                    wxy = wxs[dx] * wys[dy]
                    for dz in (0, 1):
                        t2 = ((hxy ^ hz[dz]) & (T_PER - 1)) << 1
                        w3 = wxy * wzs[dz]
                        f0 = plsc.load_gather(sub_v, [t2])
                        f1 = plsc.load_gather(sub_v, [t2 + 1])
                        acc0 = acc0 + w3 * f0
                        acc1 = acc1 + w3 * f1
            f0_v[0, sl] = acc0
            f1_v[0, sl] = acc1
            return carry2

        lax.fori_loop(0, CH_A // 16, vec_body, 0)
        pltpu.sync_copy(f0_v, featsT.at[pl.ds(2 * level, 1), pl.ds(base, CH_A)])
        pltpu.sync_copy(f1_v, featsT.at[pl.ds(2 * level + 1, 1), pl.ds(base, CH_A)])
        return carry

    lax.fori_loop(0, (P_TOTAL // 2) // CH_A, chunk_body, 0)


_gather_call = pl.kernel(
    _gather_body,
    out_type=jax.ShapeDtypeStruct((2 * L, P_TOTAL), jnp.float32),
    mesh=_MESH,
    compiler_params=_SC_PARAMS,
    scratch_types=[
        pltpu.VMEM((T_PER * F,), jnp.float32),
        pltpu.VMEM((1, CH_A), jnp.float32),
        pltpu.VMEM((1, CH_A), jnp.float32),
        pltpu.VMEM((1, CH_A), jnp.float32),
        pltpu.VMEM((1, CH_A), jnp.float32),
        pltpu.VMEM((1, CH_A), jnp.float32),
    ],
)


CH_DN = P_TOTAL // 32
ZONE_T = NPIX // 32
ZPL = ZONE_T + 16
CH_S = 1024


def _dense_body(wbase_hbm, featsT_hbm, wrep_hbm, brep_hbm,
                valsF_hbm, wsafe_hbm,
                wb_v, vals_v, fbuf, wmat_v, bvec_v, wsb, sem):
    c = lax.axis_index("c")
    s = lax.axis_index("s")
    wid = c * NS + s

    pltpu.sync_copy(wrep_hbm, wmat_v)
    pltpu.sync_copy(brep_hbm, bvec_v)
    pbase = wid * CH_DN
    pltpu.sync_copy(wbase_hbm.at[pl.ds(pbase, CH_DN)], wb_v)

    def dense_sub(sub, carry0):
        dbase = pbase + sub * CH_D
        descs = [pltpu.async_copy(
            featsT_hbm.at[pl.ds(r, 1), pl.ds(dbase, CH_D)],
            fbuf.at[pl.ds(r, 1), :], sem) for r in range(32)]
        for d in descs:
            d.wait()

        def dense_vec(i, carry):
            sl = pl.ds(i * 16, 16)
            out8 = [bvec_v[pl.ds(o * 16, 16)] for o in range(8)]
            for f in range(32):
                frow = fbuf[f, sl]
                for o in range(8):
                    out8[o] = out8[o] + wmat_v[pl.ds((f * 8 + o) * 16, 16)] * frow
            alpha = _sigmoid(out8[3])
            off = sub * CH_D + i * 16
            weight = alpha * wb_v[pl.ds(off, 16)]
            for ch in range(3):
                vals_v[pl.ds(ch * CH_DN + off, 16)] = weight * _sigmoid(out8[ch])
            for ch in range(8):
                vals_v[pl.ds((3 + ch) * CH_DN + off, 16)] = weight * out8[ch]
            vals_v[pl.ds(11 * CH_DN + off, 16)] = weight
            wsb[pl.ds(off, 16)] = weight
            return carry

        lax.fori_loop(0, CH_D // 16, dense_vec, 0)
        return carry0

    lax.fori_loop(0, CH_DN // CH_D, dense_sub, 0)
    for ch in range(12):
        pltpu.sync_copy(vals_v.at[pl.ds(ch * CH_DN, CH_DN)],
                        valsF_hbm.at[pl.ds(ch * P_TOTAL + pbase, CH_DN)])
    pltpu.sync_copy(wsb, wsafe_hbm.at[pl.ds(pbase, CH_DN)])


_dense_call = pl.kernel(
    _dense_body,
    out_type=[
        jax.ShapeDtypeStruct((12 * P_TOTAL,), jnp.float32),
        jax.ShapeDtypeStruct((P_TOTAL,), jnp.float32),
    ],
    mesh=_MESH,
    compiler_params=_SC_PARAMS,
    scratch_types=[
        pltpu.VMEM((CH_DN,), jnp.float32),
        pltpu.VMEM((12 * CH_DN,), jnp.float32),
        pltpu.VMEM((32, CH_D), jnp.float32),
        pltpu.VMEM((4096,), jnp.float32),
        pltpu.VMEM((128,), jnp.float32),
        pltpu.VMEM((CH_DN,), jnp.float32),
        pltpu.SemaphoreType.DMA,
    ],
)


def _zone_body(pix_hbm, valsF_hbm, bg_hbm, colPM_hbm, featPM_hbm,
               acc, pixc, vchunk, bg_v, colbuf, featbuf, sem):
    c = lax.axis_index("c")
    s = lax.axis_index("s")
    wid = c * NS + s
    zb = wid * ZONE_T

    pltpu.sync_copy(bg_hbm, bg_v)

    def zero_body(i, carry):
        acc[pl.ds(i * 16, 16)] = jnp.zeros((16,), jnp.float32)
        return carry

    lax.fori_loop(0, (12 * ZPL) // 16, zero_body, 0)

    def chunk_body(g, carry0):
        base = g * CH_S
        descs = [pltpu.async_copy(pix_hbm.at[pl.ds(base, CH_S)], pixc, sem)]
        for ch in range(12):
            descs.append(pltpu.async_copy(
                valsF_hbm.at[pl.ds(ch * P_TOTAL + base, CH_S)],
                vchunk.at[pl.ds(ch * CH_S, CH_S)], sem))
        for d in descs:
            d.wait()

        dump = jnp.int32(ZONE_T) + (lax.iota(jnp.int32, 16) & 7)

        def vec(i, carry):
            sl = pl.ds(i * 16, 16)
            p = pixc[sl]
            local = p - zb
            ok = (local >= 0) & (local < ZONE_T)
            li = jnp.where(ok, local, dump)
            for ch in range(12):
                v = vchunk[pl.ds(ch * CH_S + i * 16, 16)]
                plsc.addupdate_scatter(acc, [li + ch * ZPL], v)
            return carry

        lax.fori_loop(0, CH_S // 16, vec, 0)
        return carry0

    lax.fori_loop(0, P_TOTAL // CH_S, chunk_body, 0)

    lane3 = lax.iota(jnp.int32, 16) * 3
    lane8 = lax.iota(jnp.int32, 16) * 8

    def comp_sub(sub, carry0):
        rb = sub * SUBPX

        def comp_vec(i, carry):
            off = rb + i * 16
            wv = acc[pl.ds(11 * ZPL + off, 16)]
            a = jnp.clip(wv, 0.0, 1.0)
            denom = wv + 1e-8
            for ch in range(3):
                v = acc[pl.ds(ch * ZPL + off, 16)]
                r = (v / denom) * a + (1.0 - a) * bg_v[pl.ds(ch * 16, 16)]
                plsc.store_scatter(colbuf, [lane3 + (i * 48 + ch)], r)
            for ch in range(8):
                v = acc[pl.ds((3 + ch) * ZPL + off, 16)]
                plsc.store_scatter(featbuf, [lane8 + (i * 128 + ch)],
                                   (v / denom) * a)
            return carry

        lax.fori_loop(0, SUBPX // 16, comp_vec, 0)
        pltpu.sync_copy(colbuf, colPM_hbm.at[pl.ds((zb + rb) * 3, SUBPX * 3)])
        pltpu.sync_copy(featbuf, featPM_hbm.at[pl.ds((zb + rb) * 8, SUBPX * 8)])
        return carry0

    lax.fori_loop(0, ZONE_T // SUBPX, comp_sub, 0)


_zone_call = pl.kernel(
    _zone_body,
    out_type=[
        jax.ShapeDtypeStruct((NPIX * 3,), jnp.float32),
        jax.ShapeDtypeStruct((NPIX * 8,), jnp.float32),
    ],
    mesh=_MESH,
    compiler_params=_SC_PARAMS,
    scratch_types=[
        pltpu.VMEM((12 * ZPL,), jnp.float32),
        pltpu.VMEM((CH_S,), jnp.int32),
        pltpu.VMEM((12 * CH_S,), jnp.float32),
        pltpu.VMEM((48,), jnp.float32),
        pltpu.VMEM((SUBPX * 3,), jnp.float32),
        pltpu.VMEM((SUBPX * 8,), jnp.float32),
        pltpu.SemaphoreType.DMA,
    ],
)


def kernel(means3D, primitive_confidences, feature_table, linear_weights,
           linear_bias, viewmatrix, projmatrix, cam_pos, bg_color):
    meansT = means3D.T
    confT = primitive_confidences.T
    mrep = jnp.repeat(jnp.concatenate(
        [viewmatrix[2], projmatrix[0], projmatrix[1], projmatrix[3]]
    ).astype(jnp.bfloat16).astype(jnp.float32), 16)
    wrep = jnp.repeat(linear_weights.reshape(-1), 16)
    brep = jnp.repeat(linear_bias, 16)
    bg48 = jnp.repeat(bg_color, 16)
    pix, wbase = _proj_call(meansT, confT, mrep)
    featsT = _gather_call(meansT, feature_table.reshape(-1))
    valsF, wsafe = _dense_call(wbase, featsT, wrep, brep)
    colPM, featPM = _zone_call(pix, valsF, bg48)
    out_color = colPM.reshape(H, W, 3)
    out_features = featPM.reshape(H, W, 8)
    return out_color, out_features, wsafe
